# Initial kernel scaffold; baseline (speedup 1.0000x reference)
import jax
import jax.numpy as jnp
from jax.experimental import pallas as pl


def kernel(x, params):
    raise NotImplementedError("write your pallas kernel here")


# speedup vs baseline: 2.7166x; 2.7166x over previous
import functools

import jax
import jax.numpy as jnp
from jax.experimental import pallas as pl
from jax.experimental.pallas import tpu as pltpu

EPS = 1e-6
NEG = 0.2
_B, _N, _K = 8, 1024, 20


def _knn_edge_body(pts_ref, out_ref, pd_ref):
    pts = pts_ref[0]
    xx = jnp.sum(pts * pts, axis=1)
    inner = jax.lax.dot_general(
        pts, pts, (((1,), (1,)), ((), ())),
        preferred_element_type=jnp.float32,
        precision=jax.lax.Precision.DEFAULT)
    pd_ref[...] = 2.0 * inner - xx[:, None] - xx[None, :]
    iota = jax.lax.broadcasted_iota(jnp.int32, (_N, _N), 1)

    def body(k, carry):
        pd = pd_ref[...]
        m = jnp.max(pd, axis=1, keepdims=True)
        eq = pd >= m
        idx = jnp.min(jnp.where(eq, iota, _N), axis=1)
        oh = (iota == idx[:, None])
        nbr = jax.lax.dot_general(
            oh.astype(jnp.float32), pts, (((1,), (0,)), ((), ())),
            preferred_element_type=jnp.float32,
            precision=jax.lax.Precision.HIGHEST)
        out_ref[0, k] = nbr - pts
        pd_ref[...] = jnp.where(oh, -jnp.inf, pd)
        return carry

    jax.lax.fori_loop(0, _K, body, 0, unroll=False)


@functools.partial(jax.jit, static_argnums=1)
def _knn_edges(pts, c3):
    return pl.pallas_call(
        _knn_edge_body,
        grid=(_B,),
        in_specs=[pl.BlockSpec((1, _N, c3), lambda b: (b, 0, 0))],
        out_specs=pl.BlockSpec((1, _K, _N, c3), lambda b: (b, 0, 0, 0)),
        out_shape=jax.ShapeDtypeStruct((_B, _K, _N, c3), jnp.float32),
        scratch_shapes=[pltpu.VMEM((_N, _N), jnp.float32)],
    )(pts)


def _graph_feature(x):
    b, c, _, n = x.shape
    xf = x.reshape(b, c * 3, n)
    pts = jnp.transpose(xf, (0, 2, 1))
    edge = _knn_edges(pts, c * 3)
    em = edge.reshape(b, _K, n, c, 3).transpose(0, 3, 4, 2, 1)
    ctr = pts.reshape(b, n, c, 3).transpose(0, 2, 3, 1)[..., None]
    ctr = jnp.broadcast_to(ctr, em.shape)
    return jnp.concatenate([em, ctr], axis=1)


def _vn_linear(W, x):
    xm = jnp.moveaxis(x, 1, -1)
    return jnp.moveaxis(xm @ W.T, -1, 1)


def _vn_batchnorm(x, gamma, beta):
    norm = jnp.linalg.norm(x, axis=2) + EPS
    axes = (0,) + tuple(range(2, norm.ndim))
    mean = norm.mean(axis=axes, keepdims=True)
    var = jnp.var(norm, axis=axes, keepdims=True)
    shp = (1, -1) + (1,) * (norm.ndim - 2)
    nbn = (norm - mean) / jnp.sqrt(var + 1e-5) * gamma.reshape(shp) + beta.reshape(shp)
    return x / jnp.expand_dims(norm, 2) * jnp.expand_dims(nbn, 2)


def _vn_llr(x, Wf, Wd, gamma, beta):
    p = _vn_batchnorm(_vn_linear(Wf, x), gamma, beta)
    d = _vn_linear(Wd, x)
    dot = jnp.sum(p * d, axis=2, keepdims=True)
    mask = (dot >= 0).astype(x.dtype)
    d2 = jnp.sum(d * d, axis=2, keepdims=True)
    return NEG * p + (1 - NEG) * (mask * p + (1 - mask) * (p - (dot / (d2 + EPS)) * d))


def _complex_llr(x, j, Wr, Wi):
    kk = j / (jnp.linalg.norm(j, axis=-1, keepdims=True) + EPS)
    x_para = jnp.sum(x * kk, axis=-1, keepdims=True) * kk
    x_perp = x - x_para
    x_dual = jnp.cross(kk, x_perp)
    lin = lambda W, v: jnp.einsum('oc,bnci->bnoi', W, v)
    real = lin(Wr, x_perp) - lin(Wi, x_dual)
    imag = lin(Wr, x_dual) + lin(Wi, x_perp)
    dot = jnp.sum(real * imag, axis=-1, keepdims=True)
    mask = (dot >= 0).astype(x.dtype)
    d2 = jnp.sum(imag * imag, axis=-1, keepdims=True)
    out = NEG * real + (1 - NEG) * (mask * real + (1 - mask) * (real - (dot / (d2 + EPS)) * imag))
    return jnp.transpose(out, (0, 2, 3, 1))


def kernel(x, params):
    p = params
    x0 = x[:, None, :, :]
    f = _graph_feature(x0)
    f = _vn_llr(f, p['c1_Wf'], p['c1_Wd'], p['c1_g'], p['c1_b'])
    f = _vn_llr(f, p['c2_Wf'], p['c2_Wd'], p['c2_g'], p['c2_b'])
    x1 = f.mean(axis=-1)
    f = _graph_feature(x1)
    f = _vn_llr(f, p['c3_Wf'], p['c3_Wd'], p['c3_g'], p['c3_b'])
    f = _vn_llr(f, p['c4_Wf'], p['c4_Wd'], p['c4_g'], p['c4_b'])
    x2 = f.mean(axis=-1)
    f = _graph_feature(x2)
    f = _vn_llr(f, p['c5_Wf'], p['c5_Wd'], p['c5_g'], p['c5_b'])
    x3 = f.mean(axis=-1)
    x123 = jnp.concatenate([x1, x2, x3], axis=1)
    xa = jnp.transpose(_vn_linear(p['lin1_W'], x123), (0, 3, 1, 2))
    ja = jnp.transpose(_vn_linear(p['lin2_W'], x123), (0, 3, 1, 2))
    y = _complex_llr(xa, ja, p['cl1_Wr'], p['cl1_Wi'])
    comb = jnp.concatenate([jnp.transpose(xa, (0, 2, 3, 1)), y], axis=1)
    xb = jnp.transpose(_vn_linear(p['lin3_W'], comb), (0, 3, 1, 2))
    jb = jnp.transpose(_vn_linear(p['lin4_W'], comb), (0, 3, 1, 2))
    y2 = _complex_llr(xb, jb, p['cl2_Wr'], p['cl2_Wi'])
    comb = jnp.concatenate([jnp.transpose(xb, (0, 2, 3, 1)), y2], axis=1)
    xc = _vn_llr(comb, p['c6_Wf'], p['c6_Wd'], p['c6_g'], p['c6_b'])
    x_mean = jnp.broadcast_to(xc.mean(axis=-1, keepdims=True), xc.shape)
    xc = jnp.concatenate([xc, x_mean], axis=1)
    xp = xc.mean(axis=-1)
    z0 = _vn_llr(xp, p['inv1_Wf'], p['inv1_Wd'], p['inv1_g'], p['inv1_b'])
    z0 = _vn_llr(z0, p['inv2_Wf'], p['inv2_Wd'], p['inv2_g'], p['inv2_b'])
    z0 = _vn_linear(p['inv3_W'], z0)
    z0 = jnp.transpose(z0, (0, 2, 1))
    x_std = jnp.einsum('bij,bjk->bik', xp, z0)
    x1_out = x_std @ p['lin0_W'].T + p['lin0_b']
    return xp, x1_out
